# baseline (speedup 1.0000x reference)
import functools

import jax
import jax.numpy as jnp
import numpy as np
from jax import lax
from jax.experimental import pallas as pl
from jax.experimental.pallas import tpu as pltpu
from jax.experimental.pallas import tpu_sc as plsc

_BINS = 30
_BN = 2048
_NW = 16
_L = 16
_NB = 2 * _L

_LAST_EDGE = float(np.float32(1.0) + np.float32(1e-06))


_SPLIT = 2


def _ce_block_body(tgt_ref, *refs):
    out_ref = refs[-1]
    for h in range(_SPLIT):
        x = refs[h][...]
        t = tgt_ref[0, h, :]
        m = jnp.max(x, axis=1, keepdims=True)
        s = jnp.sum(jnp.exp(x - m), axis=1)
        lse = m[:, 0] + jnp.log(s)
        cols = lax.broadcasted_iota(jnp.int32, x.shape, 1)
        xt = jnp.sum(jnp.where(cols == t[:, None], x, 0.0), axis=1)
        out_ref[0, h, :] = lse - xt


def _ce_rows(x, t3):
    n, c = x.shape
    grid = n // (_BN * _SPLIT)

    def x_spec(h):
        return pl.BlockSpec((_BN, c), lambda i, h=h: (h * grid + i, 0))

    return pl.pallas_call(
        _ce_block_body,
        grid=(grid,),
        in_specs=[pl.BlockSpec((1, _SPLIT, _BN), lambda i: (i, 0, 0))]
        + [x_spec(h) for h in range(_SPLIT)],
        out_specs=pl.BlockSpec((1, _SPLIT, _BN), lambda i: (i, 0, 0)),
        out_shape=jax.ShapeDtypeStruct((grid, _SPLIT, _BN), jnp.float32),
        compiler_params=pltpu.CompilerParams(
            dimension_semantics=("arbitrary",),
        ),
    )(t3, *([x] * _SPLIT))


def _sc_hist_body(ce_hbm, cnt_hbm, sum_hbm, ce_v, cnt_v, sum_v, out_v):
    n = ce_hbm.shape[0]
    chunk = n // _NW
    wid = lax.axis_index("s")
    base = wid * chunk
    pltpu.sync_copy(ce_hbm.at[pl.ds(base, chunk)], ce_v)

    zeros = jnp.zeros((_L,), jnp.float32)
    cnt_v[pl.ds(0, _L)] = zeros
    cnt_v[pl.ds(_L, _L)] = zeros
    sum_v[pl.ds(0, _L)] = zeros
    sum_v[pl.ds(_L, _L)] = zeros

    ones = jnp.ones((_L,), jnp.float32)

    def step(i, carry):
        ce = ce_v[pl.ds(i * _L, _L)]
        g = 1.0 - jnp.exp(-ce)
        b0 = jnp.clip((g * np.float32(_BINS)).astype(jnp.int32), 0, _BINS - 1)
        lo = b0.astype(jnp.float32) / np.float32(_BINS)
        hi0 = (b0 + 1).astype(jnp.float32) / np.float32(_BINS)
        hi = jnp.where(b0 == _BINS - 1, np.float32(_LAST_EDGE), hi0)
        b = b0 + (g >= hi).astype(jnp.int32) - (g < lo).astype(jnp.int32)
        b = jnp.clip(b, 0, _BINS - 1)
        plsc.addupdate_scatter(cnt_v, [b], ones)
        plsc.addupdate_scatter(sum_v, [b], ce)
        return carry

    lax.fori_loop(0, chunk // _L, step, 0)

    out_v[pl.ds(0, _L)] = cnt_v[pl.ds(0, _L)]
    out_v[pl.ds(_L, _L)] = cnt_v[pl.ds(_L, _L)]
    pltpu.sync_copy(out_v, cnt_hbm.at[wid])
    out_v[pl.ds(0, _L)] = sum_v[pl.ds(0, _L)]
    out_v[pl.ds(_L, _L)] = sum_v[pl.ds(_L, _L)]
    pltpu.sync_copy(out_v, sum_hbm.at[wid])


@functools.lru_cache(maxsize=None)
def _sc_hist(n):
    return pl.kernel(
        _sc_hist_body,
        out_type=(
            jax.ShapeDtypeStruct((_NW, _NB), jnp.float32),
            jax.ShapeDtypeStruct((_NW, _NB), jnp.float32),
        ),
        mesh=plsc.VectorSubcoreMesh(core_axis_name="c", subcore_axis_name="s",
                                    num_cores=1),
        scratch_types=[
            pltpu.VMEM((n // _NW,), jnp.float32),
            pltpu.VMEM((_NB,), jnp.float32),
            pltpu.VMEM((_NB,), jnp.float32),
            pltpu.VMEM((_NB,), jnp.float32),
        ],
        compiler_params=pltpu.CompilerParams(needs_layout_passes=False),
    )


def _finalize_body(cnt_ref, sum_ref, out_ref):
    c = jnp.sum(cnt_ref[...], axis=0, keepdims=True)
    s = jnp.sum(sum_ref[...], axis=0, keepdims=True)
    nz = c > 0.0
    m_cnt = jnp.sum(nz.astype(jnp.float32))
    terms = jnp.where(nz, s / jnp.where(nz, c, 1.0), 0.0)
    out_ref[0, 0] = 4.0 * jnp.sum(terms) / m_cnt


def _finalize(cnt, s):
    return pl.pallas_call(
        _finalize_body,
        in_specs=[
            pl.BlockSpec(memory_space=pltpu.VMEM),
            pl.BlockSpec(memory_space=pltpu.VMEM),
        ],
        out_specs=pl.BlockSpec(memory_space=pltpu.SMEM),
        out_shape=jax.ShapeDtypeStruct((1, 1), jnp.float32),
    )(cnt, s)


def kernel(input, target):
    n, c = input.shape
    grid = n // (_BN * _SPLIT)
    t3 = (target.astype(jnp.int32)
          .reshape(_SPLIT, grid, _BN).transpose(1, 0, 2))
    ce = _ce_rows(input, t3).reshape(n)
    cnt, s = _sc_hist(n)(ce)
    return _finalize(cnt, s)[0, 0]


# speedup vs baseline: 2.6606x; 2.6606x over previous
import functools

import jax
import jax.numpy as jnp
import numpy as np
from jax import lax
from jax.experimental import pallas as pl
from jax.experimental.pallas import tpu as pltpu
from jax.experimental.pallas import tpu_sc as plsc

_BINS = 30
_BN = 2048
_NW = 16
_L = 16
_NB = 2 * _L

_LAST_EDGE = float(np.float32(1.0) + np.float32(1e-06))


def _ce_block_body(tgt_ref, x_ref, out_ref):
    x = x_ref[...]
    t = tgt_ref[0, 0, :]
    m = jnp.max(x, axis=0, keepdims=True)
    s = jnp.sum(jnp.exp(x - m), axis=0)
    lse = m[0, :] + jnp.log(s)
    rows = lax.broadcasted_iota(jnp.int32, x.shape, 0)
    xt = jnp.sum(jnp.where(rows == t[None, :], x, 0.0), axis=0)
    out_ref[0, 0, :] = lse - xt


def _ce_rows(xt, t3):
    c, n = xt.shape
    grid = n // _BN
    return pl.pallas_call(
        _ce_block_body,
        grid=(grid,),
        in_specs=[
            pl.BlockSpec((1, 1, _BN), lambda i: (i, 0, 0)),
            pl.BlockSpec((c, _BN), lambda i: (0, i)),
        ],
        out_specs=pl.BlockSpec((1, 1, _BN), lambda i: (i, 0, 0)),
        out_shape=jax.ShapeDtypeStruct((grid, 1, _BN), jnp.float32),
        compiler_params=pltpu.CompilerParams(
            dimension_semantics=("arbitrary",),
        ),
    )(t3, xt)


def _sc_hist_body(ce_hbm, cnt_hbm, sum_hbm, ce_v, cnt_v, sum_v, out_v):
    n = ce_hbm.shape[0]
    chunk = n // _NW
    wid = lax.axis_index("s")
    base = wid * chunk
    pltpu.sync_copy(ce_hbm.at[pl.ds(base, chunk)], ce_v)

    zeros = jnp.zeros((_L,), jnp.float32)
    cnt_v[pl.ds(0, _L)] = zeros
    cnt_v[pl.ds(_L, _L)] = zeros
    sum_v[pl.ds(0, _L)] = zeros
    sum_v[pl.ds(_L, _L)] = zeros

    ones = jnp.ones((_L,), jnp.float32)

    def step(i, carry):
        ce = ce_v[pl.ds(i * _L, _L)]
        g = 1.0 - jnp.exp(-ce)
        b0 = jnp.clip((g * np.float32(_BINS)).astype(jnp.int32), 0, _BINS - 1)
        lo = b0.astype(jnp.float32) / np.float32(_BINS)
        hi0 = (b0 + 1).astype(jnp.float32) / np.float32(_BINS)
        hi = jnp.where(b0 == _BINS - 1, np.float32(_LAST_EDGE), hi0)
        b = b0 + (g >= hi).astype(jnp.int32) - (g < lo).astype(jnp.int32)
        b = jnp.clip(b, 0, _BINS - 1)
        plsc.addupdate_scatter(cnt_v, [b], ones)
        plsc.addupdate_scatter(sum_v, [b], ce)
        return carry

    lax.fori_loop(0, chunk // _L, step, 0)

    out_v[pl.ds(0, _L)] = cnt_v[pl.ds(0, _L)]
    out_v[pl.ds(_L, _L)] = cnt_v[pl.ds(_L, _L)]
    pltpu.sync_copy(out_v, cnt_hbm.at[wid])
    out_v[pl.ds(0, _L)] = sum_v[pl.ds(0, _L)]
    out_v[pl.ds(_L, _L)] = sum_v[pl.ds(_L, _L)]
    pltpu.sync_copy(out_v, sum_hbm.at[wid])


@functools.lru_cache(maxsize=None)
def _sc_hist(n):
    return pl.kernel(
        _sc_hist_body,
        out_type=(
            jax.ShapeDtypeStruct((_NW, _NB), jnp.float32),
            jax.ShapeDtypeStruct((_NW, _NB), jnp.float32),
        ),
        mesh=plsc.VectorSubcoreMesh(core_axis_name="c", subcore_axis_name="s",
                                    num_cores=1),
        scratch_types=[
            pltpu.VMEM((n // _NW,), jnp.float32),
            pltpu.VMEM((_NB,), jnp.float32),
            pltpu.VMEM((_NB,), jnp.float32),
            pltpu.VMEM((_NB,), jnp.float32),
        ],
        compiler_params=pltpu.CompilerParams(needs_layout_passes=False),
    )


def _finalize_body(cnt_ref, sum_ref, out_ref):
    c = jnp.sum(cnt_ref[...], axis=0, keepdims=True)
    s = jnp.sum(sum_ref[...], axis=0, keepdims=True)
    nz = c > 0.0
    m_cnt = jnp.sum(nz.astype(jnp.float32))
    terms = jnp.where(nz, s / jnp.where(nz, c, 1.0), 0.0)
    out_ref[0, 0] = 4.0 * jnp.sum(terms) / m_cnt


def _finalize(cnt, s):
    return pl.pallas_call(
        _finalize_body,
        in_specs=[
            pl.BlockSpec(memory_space=pltpu.VMEM),
            pl.BlockSpec(memory_space=pltpu.VMEM),
        ],
        out_specs=pl.BlockSpec(memory_space=pltpu.SMEM),
        out_shape=jax.ShapeDtypeStruct((1, 1), jnp.float32),
    )(cnt, s)


def kernel(input, target):
    n, c = input.shape
    t3 = target.astype(jnp.int32).reshape(n // _BN, 1, _BN)
    ce = _ce_rows(input.T, t3).reshape(n)
    cnt, s = _sc_hist(n)(ce)
    return _finalize(cnt, s)[0, 0]


# speedup vs baseline: 2.7885x; 1.0481x over previous
import functools

import jax
import jax.numpy as jnp
import numpy as np
from jax import lax
from jax.experimental import pallas as pl
from jax.experimental.pallas import tpu as pltpu
from jax.experimental.pallas import tpu_sc as plsc

_BINS = 30
_BN = 2048
_NW = 32
_L = 16
_NB = 2 * _L

_LAST_EDGE = float(np.float32(1.0) + np.float32(1e-06))


def _ce_block_body(tgt_ref, x_ref, out_ref):
    x = x_ref[...]
    t = tgt_ref[0, 0, :]
    m = jnp.max(x, axis=0, keepdims=True)
    s = jnp.sum(jnp.exp(x - m), axis=0)
    lse = m[0, :] + jnp.log(s)
    rows = lax.broadcasted_iota(jnp.int32, x.shape, 0)
    xt = jnp.sum(jnp.where(rows == t[None, :], x, 0.0), axis=0)
    out_ref[0, 0, :] = lse - xt


def _ce_rows(xt, t3):
    c, n = xt.shape
    grid = n // _BN
    return pl.pallas_call(
        _ce_block_body,
        grid=(grid,),
        in_specs=[
            pl.BlockSpec((1, 1, _BN), lambda i: (i, 0, 0)),
            pl.BlockSpec((c, _BN), lambda i: (0, i)),
        ],
        out_specs=pl.BlockSpec((1, 1, _BN), lambda i: (i, 0, 0)),
        out_shape=jax.ShapeDtypeStruct((grid, 1, _BN), jnp.float32),
        compiler_params=pltpu.CompilerParams(
            dimension_semantics=("arbitrary",),
        ),
    )(t3, xt)


def _sc_hist_body(ce_hbm, cnt_hbm, sum_hbm, ce_v, cnt_v, sum_v, out_v):
    n = ce_hbm.shape[0]
    chunk = n // _NW
    wid = lax.axis_index("s") * 2 + lax.axis_index("c")
    base = wid * chunk
    pltpu.sync_copy(ce_hbm.at[pl.ds(base, chunk)], ce_v)

    zeros = jnp.zeros((_L,), jnp.float32)
    cnt_v[pl.ds(0, _L)] = zeros
    cnt_v[pl.ds(_L, _L)] = zeros
    sum_v[pl.ds(0, _L)] = zeros
    sum_v[pl.ds(_L, _L)] = zeros

    ones = jnp.ones((_L,), jnp.float32)

    def step(i, carry):
        ce = ce_v[pl.ds(i * _L, _L)]
        g = 1.0 - jnp.exp(-ce)
        b0 = jnp.clip((g * np.float32(_BINS)).astype(jnp.int32), 0, _BINS - 1)
        lo = b0.astype(jnp.float32) / np.float32(_BINS)
        hi0 = (b0 + 1).astype(jnp.float32) / np.float32(_BINS)
        hi = jnp.where(b0 == _BINS - 1, np.float32(_LAST_EDGE), hi0)
        b = b0 + (g >= hi).astype(jnp.int32) - (g < lo).astype(jnp.int32)
        b = jnp.clip(b, 0, _BINS - 1)
        plsc.addupdate_scatter(cnt_v, [b], ones)
        plsc.addupdate_scatter(sum_v, [b], ce)
        return carry

    lax.fori_loop(0, chunk // _L, step, 0)

    out_v[pl.ds(0, _L)] = cnt_v[pl.ds(0, _L)]
    out_v[pl.ds(_L, _L)] = cnt_v[pl.ds(_L, _L)]
    pltpu.sync_copy(out_v, cnt_hbm.at[wid])
    out_v[pl.ds(0, _L)] = sum_v[pl.ds(0, _L)]
    out_v[pl.ds(_L, _L)] = sum_v[pl.ds(_L, _L)]
    pltpu.sync_copy(out_v, sum_hbm.at[wid])


@functools.lru_cache(maxsize=None)
def _sc_hist(n):
    return pl.kernel(
        _sc_hist_body,
        out_type=(
            jax.ShapeDtypeStruct((_NW, _NB), jnp.float32),
            jax.ShapeDtypeStruct((_NW, _NB), jnp.float32),
        ),
        mesh=plsc.VectorSubcoreMesh(core_axis_name="c", subcore_axis_name="s"),
        scratch_types=[
            pltpu.VMEM((n // _NW,), jnp.float32),
            pltpu.VMEM((_NB,), jnp.float32),
            pltpu.VMEM((_NB,), jnp.float32),
            pltpu.VMEM((_NB,), jnp.float32),
        ],
        compiler_params=pltpu.CompilerParams(needs_layout_passes=False),
    )


def _finalize_body(cnt_ref, sum_ref, out_ref):
    c = jnp.sum(cnt_ref[...], axis=0, keepdims=True)
    s = jnp.sum(sum_ref[...], axis=0, keepdims=True)
    nz = c > 0.0
    m_cnt = jnp.sum(nz.astype(jnp.float32))
    terms = jnp.where(nz, s / jnp.where(nz, c, 1.0), 0.0)
    out_ref[0, 0] = 4.0 * jnp.sum(terms) / m_cnt


def _finalize(cnt, s):
    return pl.pallas_call(
        _finalize_body,
        in_specs=[
            pl.BlockSpec(memory_space=pltpu.VMEM),
            pl.BlockSpec(memory_space=pltpu.VMEM),
        ],
        out_specs=pl.BlockSpec(memory_space=pltpu.SMEM),
        out_shape=jax.ShapeDtypeStruct((1, 1), jnp.float32),
    )(cnt, s)


def kernel(input, target):
    n, c = input.shape
    t3 = target.astype(jnp.int32).reshape(n // _BN, 1, _BN)
    ce = _ce_rows(input.T, t3).reshape(n)
    cnt, s = _sc_hist(n)(ce)
    return _finalize(cnt, s)[0, 0]


# speedup vs baseline: 2.8906x; 1.0366x over previous
import functools

import jax
import jax.numpy as jnp
import numpy as np
from jax import lax
from jax.experimental import pallas as pl
from jax.experimental.pallas import tpu as pltpu
from jax.experimental.pallas import tpu_sc as plsc

_BINS = 30
_BN = 4096
_NW = 32
_L = 16
_NB = 2 * _L

_LAST_EDGE = float(np.float32(1.0) + np.float32(1e-06))


def _ce_block_body(tgt_ref, x_ref, out_ref):
    x = x_ref[...]
    t = tgt_ref[0, 0, :]
    m = jnp.max(x, axis=0, keepdims=True)
    s = jnp.sum(jnp.exp(x - m), axis=0)
    lse = m[0, :] + jnp.log(s)
    rows = lax.broadcasted_iota(jnp.int32, x.shape, 0)
    xt = jnp.sum(jnp.where(rows == t[None, :], x, 0.0), axis=0)
    out_ref[0, 0, :] = lse - xt


def _ce_rows(xt, t3):
    c, n = xt.shape
    grid = n // _BN
    return pl.pallas_call(
        _ce_block_body,
        grid=(grid,),
        in_specs=[
            pl.BlockSpec((1, 1, _BN), lambda i: (i, 0, 0)),
            pl.BlockSpec((c, _BN), lambda i: (0, i)),
        ],
        out_specs=pl.BlockSpec((1, 1, _BN), lambda i: (i, 0, 0)),
        out_shape=jax.ShapeDtypeStruct((grid, 1, _BN), jnp.float32),
        compiler_params=pltpu.CompilerParams(
            dimension_semantics=("arbitrary",),
        ),
    )(t3, xt)


def _sc_hist_body(ce_hbm, cnt_hbm, sum_hbm, ce_v, cnt_v, sum_v, out_v):
    n = ce_hbm.shape[0]
    chunk = n // _NW
    wid = lax.axis_index("s") * 2 + lax.axis_index("c")
    base = wid * chunk
    pltpu.sync_copy(ce_hbm.at[pl.ds(base, chunk)], ce_v)

    zeros = jnp.zeros((_L,), jnp.float32)
    cnt_v[pl.ds(0, _L)] = zeros
    cnt_v[pl.ds(_L, _L)] = zeros
    sum_v[pl.ds(0, _L)] = zeros
    sum_v[pl.ds(_L, _L)] = zeros

    ones = jnp.ones((_L,), jnp.float32)

    def step(i, carry):
        ce = ce_v[pl.ds(i * _L, _L)]
        g = 1.0 - jnp.exp(-ce)
        b0 = jnp.clip((g * np.float32(_BINS)).astype(jnp.int32), 0, _BINS - 1)
        lo = b0.astype(jnp.float32) / np.float32(_BINS)
        hi0 = (b0 + 1).astype(jnp.float32) / np.float32(_BINS)
        hi = jnp.where(b0 == _BINS - 1, np.float32(_LAST_EDGE), hi0)
        b = b0 + (g >= hi).astype(jnp.int32) - (g < lo).astype(jnp.int32)
        b = jnp.clip(b, 0, _BINS - 1)
        plsc.addupdate_scatter(cnt_v, [b], ones)
        plsc.addupdate_scatter(sum_v, [b], ce)
        return carry

    lax.fori_loop(0, chunk // _L, step, 0)

    out_v[pl.ds(0, _L)] = cnt_v[pl.ds(0, _L)]
    out_v[pl.ds(_L, _L)] = cnt_v[pl.ds(_L, _L)]
    pltpu.sync_copy(out_v, cnt_hbm.at[wid])
    out_v[pl.ds(0, _L)] = sum_v[pl.ds(0, _L)]
    out_v[pl.ds(_L, _L)] = sum_v[pl.ds(_L, _L)]
    pltpu.sync_copy(out_v, sum_hbm.at[wid])


@functools.lru_cache(maxsize=None)
def _sc_hist(n):
    return pl.kernel(
        _sc_hist_body,
        out_type=(
            jax.ShapeDtypeStruct((_NW, _NB), jnp.float32),
            jax.ShapeDtypeStruct((_NW, _NB), jnp.float32),
        ),
        mesh=plsc.VectorSubcoreMesh(core_axis_name="c", subcore_axis_name="s"),
        scratch_types=[
            pltpu.VMEM((n // _NW,), jnp.float32),
            pltpu.VMEM((_NB,), jnp.float32),
            pltpu.VMEM((_NB,), jnp.float32),
            pltpu.VMEM((_NB,), jnp.float32),
        ],
        compiler_params=pltpu.CompilerParams(needs_layout_passes=False),
    )


def _finalize_body(cnt_ref, sum_ref, out_ref):
    c = jnp.sum(cnt_ref[...], axis=0, keepdims=True)
    s = jnp.sum(sum_ref[...], axis=0, keepdims=True)
    nz = c > 0.0
    m_cnt = jnp.sum(nz.astype(jnp.float32))
    terms = jnp.where(nz, s / jnp.where(nz, c, 1.0), 0.0)
    out_ref[0, 0] = 4.0 * jnp.sum(terms) / m_cnt


def _finalize(cnt, s):
    return pl.pallas_call(
        _finalize_body,
        in_specs=[
            pl.BlockSpec(memory_space=pltpu.VMEM),
            pl.BlockSpec(memory_space=pltpu.VMEM),
        ],
        out_specs=pl.BlockSpec(memory_space=pltpu.SMEM),
        out_shape=jax.ShapeDtypeStruct((1, 1), jnp.float32),
    )(cnt, s)


def kernel(input, target):
    n, c = input.shape
    t3 = target.astype(jnp.int32).reshape(n // _BN, 1, _BN)
    ce = _ce_rows(input.T, t3).reshape(n)
    cnt, s = _sc_hist(n)(ce)
    return _finalize(cnt, s)[0, 0]
